# baseline (speedup 1.0000x reference)
import jax
import jax.numpy as jnp
from jax.experimental import pallas as pl
from jax.experimental.pallas import tpu as pltpu

_B = 2048
_D = 128
_NCLS = 256
_MARGIN = 1.0
_BLK = 1024
_NSTEPS = _B // _BLK
_P = 65536.0


def _triplet_kernel(eb_ref, et_ref, labc_ref, labr_ref, sum_ref, cnt_ref,
                    rhs_ref, sq_ref):
    i = pl.program_id(0)
    labr = labr_ref[...]

    @pl.when(i == 0)
    def _():
        et = et_ref[...]
        sq_ref[...] = jnp.sum(et * et, axis=0, keepdims=True)
        rhs_ref[0:_D, :] = et
        cls = jax.lax.broadcasted_iota(jnp.int32, (_NCLS, 1), 0)
        oh = jnp.where(cls == labr, _P, 0.0)
        rhs_ref[_D:, :] = oh
        ncls = jnp.sum(oh, axis=1, keepdims=True) * jnp.float32(1.0 / _P)
        s1 = jnp.sum(ncls * ncls, keepdims=True)
        cnt_ref[...] = (0.5 * (s1 - jnp.float32(_B))).astype(jnp.int32)
        sum_ref[...] = jnp.zeros_like(sum_ref)

    labc = labc_ref[...]
    ebm2 = eb_ref[...] * jnp.float32(-2.0)
    cls2 = jax.lax.broadcasted_iota(jnp.int32, (1, _NCLS), 1)
    ohb = jnp.where(labc == cls2, 1.0, 0.0)
    lhs = jnp.concatenate([ebm2, ohb], axis=1)
    hm = sq_ref[...] + jnp.dot(lhs, rhs_ref[...],
                               preferred_element_type=jnp.float32)

    neg = jnp.min(hm, axis=1, keepdims=True)
    neg = jnp.where(neg < jnp.float32(0.5 * _P), neg, hm[:, 0:1] - _P)
    negt = neg + jnp.float32(_P - _MARGIN)

    colv = jax.lax.broadcasted_iota(jnp.int32, (1, _B), 1)
    rowv = i * _BLK + jax.lax.broadcasted_iota(jnp.int32, (_BLK, 1), 0)
    pos = jnp.logical_and(labc == labr, colv > rowv)

    x = jnp.maximum(hm - negt, 0.0)
    sum_ref[...] += jnp.sum(jnp.where(pos, x, 0.0), keepdims=True)

    @pl.when(i == _NSTEPS - 1)
    def _():
        sum_ref[...] = sum_ref[...] / cnt_ref[...].astype(jnp.float32)


def kernel(embeddings, target):
    et = embeddings.T
    labc = target.reshape(_B, 1)
    labr = target.reshape(1, _B)
    out_sum, out_cnt = pl.pallas_call(
        _triplet_kernel,
        grid=(_NSTEPS,),
        in_specs=[
            pl.BlockSpec((_BLK, _D), lambda i: (i, 0)),
            pl.BlockSpec((_D, _B), lambda i: (0, 0)),
            pl.BlockSpec((_BLK, 1), lambda i: (i, 0)),
            pl.BlockSpec((1, _B), lambda i: (0, 0)),
        ],
        out_specs=[
            pl.BlockSpec((1, 1), lambda i: (0, 0)),
            pl.BlockSpec((1, 1), lambda i: (0, 0)),
        ],
        out_shape=[
            jax.ShapeDtypeStruct((1, 1), jnp.float32),
            jax.ShapeDtypeStruct((1, 1), jnp.int32),
        ],
        scratch_shapes=[
            pltpu.VMEM((_D + _NCLS, _B), jnp.float32),
            pltpu.VMEM((1, _B), jnp.float32),
        ],
    )(embeddings, et, labc, labr)
    return (out_sum[0, 0], out_cnt[0, 0])


# speedup vs baseline: 1.1302x; 1.1302x over previous
import jax
import jax.numpy as jnp
from jax.experimental import pallas as pl
from jax.experimental.pallas import tpu as pltpu

_B = 2048
_D = 128
_NCLS = 256
_MARGIN = 1.0
_BLK = 1024
_NSTEPS = _B // _BLK


def _triplet_kernel(eb_ref, et_ref, labc_ref, labr_ref, sum_ref, cnt_ref,
                    sq_ref):
    i = pl.program_id(0)
    et = et_ref[...]
    labr = labr_ref[...]

    @pl.when(i == 0)
    def _():
        sq_ref[...] = jnp.sum(et * et, axis=0, keepdims=True)
        cls = jax.lax.broadcasted_iota(jnp.int32, (_NCLS, 1), 0)
        ohc = jnp.where(cls == labr, 1.0, 0.0)
        ncls = jnp.sum(ohc, axis=1, keepdims=True)
        s1 = jnp.sum(ncls * ncls, keepdims=True)
        cnt_ref[...] = (0.5 * (s1 - jnp.float32(_B))).astype(jnp.int32)
        sum_ref[...] = jnp.zeros_like(sum_ref)

    ebm2 = eb_ref[...] * jnp.float32(-2.0)
    h = sq_ref[...] + jnp.dot(ebm2, et,
                              preferred_element_type=jnp.float32)

    labc = labc_ref[...]
    eqm = labc == labr

    inf = jnp.float32(jnp.inf)
    neg = jnp.min(jnp.where(eqm, inf, h), axis=1, keepdims=True)
    neg = jnp.where(neg < inf, neg, h[:, 0:1])
    negt = neg - _MARGIN

    colv = jax.lax.broadcasted_iota(jnp.int32, (1, _B), 1)
    rowv = i * _BLK + jax.lax.broadcasted_iota(jnp.int32, (_BLK, 1), 0)
    pos = jnp.logical_and(eqm, colv > rowv)

    x = jnp.maximum(h - negt, 0.0)
    sum_ref[...] += jnp.sum(jnp.where(pos, x, 0.0), keepdims=True)

    @pl.when(i == _NSTEPS - 1)
    def _():
        sum_ref[...] = sum_ref[...] / cnt_ref[...].astype(jnp.float32)


def kernel(embeddings, target):
    et = embeddings.T
    labc = target.reshape(_B, 1)
    labr = target.reshape(1, _B)
    out_sum, out_cnt = pl.pallas_call(
        _triplet_kernel,
        grid=(_NSTEPS,),
        in_specs=[
            pl.BlockSpec((_BLK, _D), lambda i: (i, 0)),
            pl.BlockSpec((_D, _B), lambda i: (0, 0)),
            pl.BlockSpec((_BLK, 1), lambda i: (i, 0)),
            pl.BlockSpec((1, _B), lambda i: (0, 0)),
        ],
        out_specs=[
            pl.BlockSpec((1, 1), lambda i: (0, 0)),
            pl.BlockSpec((1, 1), lambda i: (0, 0)),
        ],
        out_shape=[
            jax.ShapeDtypeStruct((1, 1), jnp.float32),
            jax.ShapeDtypeStruct((1, 1), jnp.int32),
        ],
        scratch_shapes=[pltpu.VMEM((1, _B), jnp.float32)],
    )(embeddings, et, labc, labr)
    return (out_sum[0, 0], out_cnt[0, 0])


# speedup vs baseline: 1.1820x; 1.0459x over previous
import jax
import jax.numpy as jnp
from jax.experimental import pallas as pl
from jax.experimental.pallas import tpu as pltpu

_B = 2048
_D = 128
_NCLS = 256
_MARGIN = 1.0
_BLK = 1024
_NSTEPS = _B // _BLK


def _triplet_kernel(eb_ref, e_ref, labc_ref, labr_ref, sum_ref, cnt_ref,
                    sq_ref):
    i = pl.program_id(0)
    e = e_ref[...]
    labr = labr_ref[...]

    @pl.when(i == 0)
    def _():
        sqc = jnp.sum(e * e, axis=1, keepdims=True)
        sq_ref[...] = sqc.T
        cls = jax.lax.broadcasted_iota(jnp.int32, (_NCLS, 1), 0)
        ohc = jnp.where(cls == labr, 1.0, 0.0)
        ncls = jnp.sum(ohc, axis=1, keepdims=True)
        s1 = jnp.sum(ncls * ncls, keepdims=True)
        cnt_ref[...] = (0.5 * (s1 - jnp.float32(_B))).astype(jnp.int32)
        sum_ref[...] = jnp.zeros_like(sum_ref)

    ebm2 = eb_ref[...] * jnp.float32(-2.0)
    g = jax.lax.dot_general(ebm2, e, (((1,), (1,)), ((), ())),
                            preferred_element_type=jnp.float32)
    h = sq_ref[...] + g

    labc = labc_ref[...]
    eqm = labc == labr

    inf = jnp.float32(jnp.inf)
    neg = jnp.min(jnp.where(eqm, inf, h), axis=1, keepdims=True)
    neg = jnp.where(neg < inf, neg, h[:, 0:1])
    negt = neg - _MARGIN

    colv = jax.lax.broadcasted_iota(jnp.int32, (1, _B), 1)
    rowv = i * _BLK + jax.lax.broadcasted_iota(jnp.int32, (_BLK, 1), 0)
    pos = jnp.logical_and(eqm, colv > rowv)

    x = jnp.maximum(h - negt, 0.0)
    sum_ref[...] += jnp.sum(jnp.where(pos, x, 0.0), keepdims=True)

    @pl.when(i == _NSTEPS - 1)
    def _():
        sum_ref[...] = sum_ref[...] / cnt_ref[...].astype(jnp.float32)


def kernel(embeddings, target):
    labc = target.reshape(_B, 1)
    labr = target.reshape(1, _B)
    out_sum, out_cnt = pl.pallas_call(
        _triplet_kernel,
        grid=(_NSTEPS,),
        in_specs=[
            pl.BlockSpec((_BLK, _D), lambda i: (i, 0)),
            pl.BlockSpec((_B, _D), lambda i: (0, 0)),
            pl.BlockSpec((_BLK, 1), lambda i: (i, 0)),
            pl.BlockSpec((1, _B), lambda i: (0, 0)),
        ],
        out_specs=[
            pl.BlockSpec((1, 1), lambda i: (0, 0)),
            pl.BlockSpec((1, 1), lambda i: (0, 0)),
        ],
        out_shape=[
            jax.ShapeDtypeStruct((1, 1), jnp.float32),
            jax.ShapeDtypeStruct((1, 1), jnp.int32),
        ],
        scratch_shapes=[pltpu.VMEM((1, _B), jnp.float32)],
    )(embeddings, embeddings, labc, labr)
    return (out_sum[0, 0], out_cnt[0, 0])


# speedup vs baseline: 1.6245x; 1.3743x over previous
import jax
import jax.numpy as jnp
from jax.experimental import pallas as pl
from jax.experimental.pallas import tpu as pltpu

_B = 2048
_D = 128
_DA = _D + 8
_NCLS = 256
_MARGIN = 1.0
_BLK = 1024


def _aug_lhs(eb):
    sqb = jnp.sum(eb * eb, axis=1, keepdims=True)
    ones = jnp.ones_like(sqb)
    zpad = jnp.zeros((eb.shape[0], _DA - _D - 2), jnp.float32)
    return jnp.concatenate([eb * jnp.float32(-2.0), ones, sqb, zpad], axis=1)


def _dist(lhs, rhs):
    return jax.lax.dot_general(lhs, rhs, (((1,), (1,)), ((), ())),
                               preferred_element_type=jnp.float32)


def _triplet_kernel(eb_ref, e_ref, labc_ref, labr_ref, sum_ref, cnt_ref,
                    rhs_ref, cmin_ref, fb_ref):
    i = pl.program_id(0)
    labr = labr_ref[...]
    labc = labc_ref[...]
    inf = jnp.float32(jnp.inf)
    lhs = _aug_lhs(eb_ref[...])

    @pl.when(i == 0)
    def _():
        e = e_ref[...]
        rhs_ref[:, 0:_D] = e
        rhs_ref[:, _D:_D + 1] = jnp.sum(e * e, axis=1, keepdims=True)
        rhs_ref[:, _D + 1:_D + 2] = jnp.ones((_B, 1), jnp.float32)
        rhs_ref[:, _D + 2:] = jnp.zeros((_B, _DA - _D - 2), jnp.float32)
        cls = jax.lax.broadcasted_iota(jnp.int32, (_NCLS, 1), 0)
        ohc = jnp.where(cls == labr, 1.0, 0.0)
        ncls = jnp.sum(ohc, axis=1, keepdims=True)
        s1 = jnp.sum(ncls * ncls, keepdims=True)
        cnt_ref[...] = (0.5 * (s1 - jnp.float32(_B))).astype(jnp.int32)

        dist = _dist(lhs, rhs_ref[...])
        eqm = labc == labr
        mh = jnp.where(eqm, inf, dist)
        neg = jnp.min(mh, axis=1, keepdims=True)
        cmin_ref[...] = jnp.min(mh[:, _BLK:], axis=0, keepdims=True)
        fb_ref[...] = dist[0:1, _BLK:]
        neg = jnp.where(neg < inf, neg, dist[:, 0:1])
        negt = neg - _MARGIN
        x = jnp.maximum(dist - negt, 0.0)
        colv = jax.lax.broadcasted_iota(jnp.int32, (1, _BLK), 1)
        rowv = jax.lax.broadcasted_iota(jnp.int32, (_BLK, 1), 0)
        posl = jnp.logical_and(eqm[:, 0:_BLK], colv > rowv)
        sl = jnp.sum(jnp.where(posl, x[:, 0:_BLK], 0.0), keepdims=True)
        su = jnp.sum(jnp.where(eqm[:, _BLK:], x[:, _BLK:], 0.0),
                     keepdims=True)
        sum_ref[...] = sl + su

    @pl.when(i == 1)
    def _():
        dist = _dist(lhs, rhs_ref[_BLK:, :])
        labru = labr[:, _BLK:]
        eqm = labc == labru
        mh = jnp.where(eqm, inf, dist)
        neg = jnp.minimum(jnp.min(mh, axis=1, keepdims=True),
                          cmin_ref[...].T)
        neg = jnp.where(neg < inf, neg, fb_ref[...].T)
        negt = neg - _MARGIN
        x = jnp.maximum(dist - negt, 0.0)
        colv = jax.lax.broadcasted_iota(jnp.int32, (1, _BLK), 1)
        rowv = jax.lax.broadcasted_iota(jnp.int32, (_BLK, 1), 0)
        pos = jnp.logical_and(eqm, colv > rowv)
        total = sum_ref[...] + jnp.sum(jnp.where(pos, x, 0.0), keepdims=True)
        sum_ref[...] = total / cnt_ref[...].astype(jnp.float32)


def kernel(embeddings, target):
    labc = target.reshape(_B, 1)
    labr = target.reshape(1, _B)
    out_sum, out_cnt = pl.pallas_call(
        _triplet_kernel,
        grid=(2,),
        in_specs=[
            pl.BlockSpec((_BLK, _D), lambda i: (i, 0)),
            pl.BlockSpec((_B, _D), lambda i: (0, 0)),
            pl.BlockSpec((_BLK, 1), lambda i: (i, 0)),
            pl.BlockSpec((1, _B), lambda i: (0, 0)),
        ],
        out_specs=[
            pl.BlockSpec((1, 1), lambda i: (0, 0)),
            pl.BlockSpec((1, 1), lambda i: (0, 0)),
        ],
        out_shape=[
            jax.ShapeDtypeStruct((1, 1), jnp.float32),
            jax.ShapeDtypeStruct((1, 1), jnp.int32),
        ],
        scratch_shapes=[
            pltpu.VMEM((_B, _DA), jnp.float32),
            pltpu.VMEM((1, _BLK), jnp.float32),
            pltpu.VMEM((1, _BLK), jnp.float32),
        ],
    )(embeddings, embeddings, labc, labr)
    return (out_sum[0, 0], out_cnt[0, 0])


# speedup vs baseline: 1.6929x; 1.0421x over previous
import jax
import jax.numpy as jnp
from jax.experimental import pallas as pl
from jax.experimental.pallas import tpu as pltpu

_B = 2048
_D = 128
_DA = _D + 8
_NCLS = 256
_MARGIN = 1.0
_BLK = 1024


def _dist(lhs, rhs):
    return jax.lax.dot_general(lhs, rhs, (((1,), (1,)), ((), ())),
                               preferred_element_type=jnp.float32)


def _triplet_kernel(e_ref, labc_ref, labr_ref, sum_ref, cnt_ref,
                    lhs_ref, rhs_ref, cmin_ref, fb_ref):
    i = pl.program_id(0)
    labr = labr_ref[...]
    labc = labc_ref[...]
    inf = jnp.float32(jnp.inf)

    @pl.when(i == 0)
    def _():
        e = e_ref[...]
        sq = jnp.sum(e * e, axis=1, keepdims=True)
        one = jnp.ones((_B, 1), jnp.float32)
        zp = jnp.zeros((_B, _DA - _D - 2), jnp.float32)
        lhs_ref[:, 0:_D] = e
        lhs_ref[:, _D:_D + 1] = sq
        lhs_ref[:, _D + 1:_D + 2] = one
        lhs_ref[:, _D + 2:] = zp
        rhs_ref[:, 0:_D] = e * jnp.float32(-2.0)
        rhs_ref[:, _D:_D + 1] = one
        rhs_ref[:, _D + 1:_D + 2] = sq
        rhs_ref[:, _D + 2:] = zp
        cls = jax.lax.broadcasted_iota(jnp.int32, (_NCLS, 1), 0)
        ohc = jnp.where(cls == labr, 1.0, 0.0)
        ncls = jnp.sum(ohc, axis=1, keepdims=True)
        s1 = jnp.sum(ncls * ncls, keepdims=True)
        cnt_ref[...] = (0.5 * (s1 - jnp.float32(_B))).astype(jnp.int32)

        dist = _dist(lhs_ref[0:_BLK, :], rhs_ref[...])
        eqm = labc == labr
        mh = jnp.where(eqm, inf, dist)
        neg = jnp.min(mh, axis=1, keepdims=True)
        cmin_ref[...] = jnp.min(mh[:, _BLK:], axis=0, keepdims=True)
        fb_ref[...] = dist[0:1, _BLK:]
        neg = jnp.where(neg < inf, neg, dist[:, 0:1])
        negt = neg - _MARGIN
        x = jnp.maximum(dist - negt, 0.0)
        colv = jax.lax.broadcasted_iota(jnp.int32, (1, _BLK), 1)
        rowv = jax.lax.broadcasted_iota(jnp.int32, (_BLK, 1), 0)
        posl = jnp.logical_and(eqm[:, 0:_BLK], colv > rowv)
        sl = jnp.sum(jnp.where(posl, x[:, 0:_BLK], 0.0), keepdims=True)
        su = jnp.sum(jnp.where(eqm[:, _BLK:], x[:, _BLK:], 0.0),
                     keepdims=True)
        sum_ref[...] = sl + su

    @pl.when(i == 1)
    def _():
        dist = _dist(lhs_ref[_BLK:, :], rhs_ref[_BLK:, :])
        labru = labr[:, _BLK:]
        eqm = labc == labru
        mh = jnp.where(eqm, inf, dist)
        neg = jnp.minimum(jnp.min(mh, axis=1, keepdims=True),
                          cmin_ref[...].T)
        neg = jnp.where(neg < inf, neg, fb_ref[...].T)
        negt = neg - _MARGIN
        x = jnp.maximum(dist - negt, 0.0)
        colv = jax.lax.broadcasted_iota(jnp.int32, (1, _BLK), 1)
        rowv = jax.lax.broadcasted_iota(jnp.int32, (_BLK, 1), 0)
        pos = jnp.logical_and(eqm, colv > rowv)
        total = sum_ref[...] + jnp.sum(jnp.where(pos, x, 0.0), keepdims=True)
        sum_ref[...] = total / cnt_ref[...].astype(jnp.float32)


def kernel(embeddings, target):
    labc = target.reshape(_B, 1)
    labr = target.reshape(1, _B)
    out_sum, out_cnt = pl.pallas_call(
        _triplet_kernel,
        grid=(2,),
        in_specs=[
            pl.BlockSpec((_B, _D), lambda i: (0, 0)),
            pl.BlockSpec((_BLK, 1), lambda i: (i, 0)),
            pl.BlockSpec((1, _B), lambda i: (0, 0)),
        ],
        out_specs=[
            pl.BlockSpec((1, 1), lambda i: (0, 0)),
            pl.BlockSpec((1, 1), lambda i: (0, 0)),
        ],
        out_shape=[
            jax.ShapeDtypeStruct((1, 1), jnp.float32),
            jax.ShapeDtypeStruct((1, 1), jnp.int32),
        ],
        scratch_shapes=[
            pltpu.VMEM((_B, _DA), jnp.float32),
            pltpu.VMEM((_B, _DA), jnp.float32),
            pltpu.VMEM((1, _BLK), jnp.float32),
            pltpu.VMEM((1, _BLK), jnp.float32),
        ],
    )(embeddings, labc, labr)
    return (out_sum[0, 0], out_cnt[0, 0])
